# baseline (speedup 1.0000x reference)
import jax
import jax.numpy as jnp
from jax import lax
from jax.experimental import pallas as pl
from jax.experimental.pallas import tpu as pltpu
from jax.experimental.pallas import tpu_sc as plsc

N = 10000
NP = 10240
E = 160000
EP = 163840
F = 128
FH = 64
C0 = 100
CP = 128
NCLASS = 40
NS = 16
NC = 2
ECB = 128
ECA = 64
NCHB = EP // ECB
NCHA = EP // ECA
RT = NP // NS

_f32 = jnp.float32
_i32 = jnp.int32


def _mesh():
    return plsc.VectorSubcoreMesh(core_axis_name="c", subcore_axis_name="s",
                                  num_cores=NC, num_subcores=NS)


def _prep_body(x_ref, xcov_ref, g_ref, b_ref, p0w_ref, p0c_ref, fc_ref,
               x0_ref, s0_ref, gp_ref, loss_ref):
    x = x_ref[...]
    mean = jnp.mean(x, axis=0, keepdims=True)
    var = jnp.mean((x - mean) ** 2, axis=0, keepdims=True)
    x0 = (x - mean) * lax.rsqrt(var + 1e-5) * g_ref[...] + b_ref[...]
    x0_ref[...] = x0

    xcov = xcov_ref[...]
    h0 = jnp.tanh(jnp.dot(xcov, p0w_ref[...], preferred_element_type=_f32))
    logits = lax.dot_general(h0, p0c_ref[...], (((1,), (1,)), ((), ())),
                             preferred_element_type=_f32)
    col = lax.broadcasted_iota(_i32, (N, CP), 1)
    lm = jnp.where(col < C0, logits, -1e30)
    m = jnp.max(lm, axis=1, keepdims=True)
    e = jnp.exp(lm - m)
    s0 = e / jnp.sum(e, axis=1, keepdims=True)
    s0_ref[...] = s0

    ones_n = jnp.ones((N, 1), _f32)
    denom = lax.dot_general(s0, ones_n, (((0,), (0,)), ((), ())),
                            preferred_element_type=_f32)
    sc_t = lax.dot_general(s0, xcov, (((0,), (0,)), ((), ())),
                           preferred_element_type=_f32)
    xc1c = sc_t * (1.0 / (denom + 1e-8))
    xcov2 = jnp.sum(xc1c, axis=0, keepdims=True) / (C0 + 1e-8)
    row = lax.broadcasted_iota(_i32, (CP, F), 0)
    corr1 = jnp.abs(xc1c - xcov2)
    loss_ref[...] = jnp.sum(jnp.where(row < C0, corr1, 0.0),
                            keepdims=True).reshape(1, 1) / (C0 * F)
    z = fc_ref[...] - corr1
    gain = 1.0 + jnp.tanh(0.5 * z)
    g2 = gain * gain
    gp_ref[...] = jnp.where(row < C0, g2, 0.0)


def _tc_prep(x, x_cov, bn_gamma, bn_beta, p0_W, p0_Cp, fc):
    return pl.pallas_call(
        _prep_body,
        out_shape=(
            jax.ShapeDtypeStruct((N, F), _f32),
            jax.ShapeDtypeStruct((N, CP), _f32),
            jax.ShapeDtypeStruct((CP, F), _f32),
            jax.ShapeDtypeStruct((1, 1), _f32),
        ),
    )(x, x_cov, bn_gamma, bn_beta, p0_W, p0_Cp, fc)


def _edges_body(s0_hbm, esrc_hbm, edst_hbm, p_hbm,
                si0, si1, di0, di1, ga0, ga1, gb0, gb1, *sems):
    cid = lax.axis_index("c")
    sid = lax.axis_index("s")
    wid = cid * NS + sid
    nw = NC * NS
    si = (si0, si1)
    di = (di0, di1)
    ga = (ga0, ga1)
    gb = (gb0, gb1)

    def e0(k):
        return pl.multiple_of((wid + nw * k) * ECB, ECB)

    def step(t, carry):
        k0 = 2 * t
        ld = []
        for u in (0, 1):
            ld.append((
                pltpu.async_copy(
                    esrc_hbm.at[pl.ds(e0(k0 + u), ECB)], si[u], sems[4 * u]),
                pltpu.async_copy(
                    edst_hbm.at[pl.ds(e0(k0 + u), ECB)], di[u],
                    sems[4 * u + 1]),
            ))

        gs = []
        for u in (0, 1):
            ld[u][0].wait()
            ld[u][1].wait()
            gs.append((
                pltpu.async_copy(s0_hbm.at[si[u]], ga[u], sems[4 * u + 2]),
                pltpu.async_copy(s0_hbm.at[di[u]], gb[u], sems[4 * u + 3]),
            ))

        for u in (0, 1):
            gs[u][0].wait()
            gs[u][1].wait()
            gau = ga[u]
            gbu = gb[u]

            @plsc.parallel_loop(0, ECB)
            def _mul(r):
                for j in range(CP // 16):
                    sl = pl.ds(j * 16, 16)
                    gau[r, sl] = gau[r, sl] * gbu[r, sl]

            pltpu.sync_copy(gau, p_hbm.at[pl.ds(e0(k0 + u), ECB)])

        return carry

    lax.fori_loop(0, NCHB // nw // 2, step, 0)


def _sc_edges(s0p, esrc, edst):
    f = pl.kernel(
        _edges_body,
        out_type=jax.ShapeDtypeStruct((EP, CP), _f32),
        mesh=_mesh(),
        scratch_types=[
            pltpu.VMEM((ECB,), _i32),
            pltpu.VMEM((ECB,), _i32),
            pltpu.VMEM((ECB,), _i32),
            pltpu.VMEM((ECB,), _i32),
            pltpu.VMEM((ECB, CP), _f32),
            pltpu.VMEM((ECB, CP), _f32),
            pltpu.VMEM((ECB, CP), _f32),
            pltpu.VMEM((ECB, CP), _f32),
        ] + [pltpu.SemaphoreType.DMA] * 8,
    )
    return f(s0p, esrc, edst)


def _exp_body(p_ref, gp_ref, adj_ref, exa_ref, exb_ref):
    v = jnp.dot(p_ref[...], gp_ref[...], preferred_element_type=_f32)
    e = jnp.exp(v + adj_ref[...])
    exa_ref[...] = e[:, :FH]
    exb_ref[...] = e[:, FH:]


def _tc_exp(p, gp, adj2d):
    blk = 1280
    return pl.pallas_call(
        _exp_body,
        grid=(EP // blk,),
        in_specs=[
            pl.BlockSpec((blk, CP), lambda i: (i, 0)),
            pl.BlockSpec((CP, F), lambda i: (0, 0)),
            pl.BlockSpec((blk, 1), lambda i: (i, 0)),
        ],
        out_specs=(
            pl.BlockSpec((blk, FH), lambda i: (i, 0)),
            pl.BlockSpec((blk, FH), lambda i: (i, 0)),
        ),
        out_shape=(
            jax.ShapeDtypeStruct((EP, FH), _f32),
            jax.ShapeDtypeStruct((EP, FH), _f32),
        ),
    )(p, gp, adj2d)


def _edge_stream(cid, sid, esrc_hbm, edst_hbm, exa_hbm, exb_hbm, table_hbm,
                 si, di, exv, gx, sems, process):
    def e0(k):
        return pl.multiple_of((sid + NS * k) * ECA, ECA)

    def step(t, carry):
        k0 = 2 * t
        ld = []
        for u in (0, 1):
            ld.append((
                pltpu.async_copy(
                    esrc_hbm.at[pl.ds(e0(k0 + u), ECA)], si[u], sems[4 * u]),
                pltpu.async_copy(
                    edst_hbm.at[pl.ds(e0(k0 + u), ECA)], di[u],
                    sems[4 * u + 1]),
            ))

            @pl.when(cid == 0)
            def _():
                pltpu.async_copy(exa_hbm.at[pl.ds(e0(k0 + u), ECA)], exv[u],
                                 sems[4 * u + 2])

            @pl.when(cid == 1)
            def _():
                pltpu.async_copy(exb_hbm.at[pl.ds(e0(k0 + u), ECA)], exv[u],
                                 sems[4 * u + 2])

        gs = []
        for u in (0, 1):
            ld[u][0].wait()
            ld[u][1].wait()
            gs.append(pltpu.async_copy(table_hbm.at[di[u]], gx[u],
                                       sems[4 * u + 3]))

        for u in (0, 1):
            pltpu.make_async_copy(exa_hbm.at[pl.ds(0, ECA)], exv[u],
                                  sems[4 * u + 2]).wait()
            gs[u].wait()
            process(u)

        return carry

    lax.fori_loop(0, NCHA // NS // 2, step, 0)


def _agg1_body(x0_hbm, esrc_hbm, edst_hbm, exa_hbm, exb_hbm,
               xc1a_hbm, xc1b_hbm, ra_hbm, rb_hbm,
               acc_sp, si0, si1, di0, di1, exv0, exv1, gx0, gx1, mg, *sems):
    cid = lax.axis_index("c")
    sid = lax.axis_index("s")
    r0 = sid * RT
    si = (si0, si1)
    di = (di0, di1)
    exv = (exv0, exv1)
    gx = (gx0, gx1)

    @plsc.parallel_loop(0, ECA)
    def _z(r):
        for j in range(F // 16):
            mg[r, pl.ds(j * 16, 16)] = jnp.zeros((16,), _f32)

    for b in range(RT // ECA):
        pltpu.sync_copy(mg, acc_sp.at[pl.ds(r0 + b * ECA, ECA)])
    plsc.subcore_barrier()

    def process(cur):
        exc = exv[cur]
        gxc = gx[cur]

        @pl.when(cid == 0)
        def _():
            @plsc.parallel_loop(0, ECA)
            def _mul(r):
                for j in range(FH // 16):
                    sl = pl.ds(j * 16, 16)
                    v = exc[r, sl]
                    mg[r, sl] = v
                    mg[r, pl.ds(FH + j * 16, 16)] = v * gxc[r, sl]

        @pl.when(cid == 1)
        def _():
            @plsc.parallel_loop(0, ECA)
            def _mul(r):
                for j in range(FH // 16):
                    sl = pl.ds(j * 16, 16)
                    slh = pl.ds(FH + j * 16, 16)
                    v = exc[r, sl]
                    mg[r, sl] = v
                    mg[r, slh] = v * gxc[r, slh]

        pltpu.sync_copy(mg, acc_sp.at[si[cur]], add=True)

    _edge_stream(cid, sid, esrc_hbm, edst_hbm, exa_hbm, exb_hbm, x0_hbm,
                 si, di, exv, gx, sems, process)
    plsc.subcore_barrier()

    for b in range(RT // ECA):
        rr = r0 + b * ECA
        pltpu.sync_copy(acc_sp.at[pl.ds(rr, ECA)], mg)
        pltpu.sync_copy(x0_hbm.at[pl.ds(rr, ECA)], gx0)

        @pl.when(cid == 0)
        def _():
            @plsc.parallel_loop(0, ECA)
            def _fin(r):
                for j in range(FH // 16):
                    sl = pl.ds(j * 16, 16)
                    rcp = 1.0 / (mg[r, sl] + 1e-16)
                    exv0[r, sl] = rcp
                    exv1[r, sl] = (rcp * mg[r, pl.ds(FH + j * 16, 16)]
                                   + gx0[r, sl])

            pltpu.sync_copy(exv0, ra_hbm.at[pl.ds(rr, ECA)])
            pltpu.sync_copy(exv1, xc1a_hbm.at[pl.ds(rr, ECA)])

        @pl.when(cid == 1)
        def _():
            @plsc.parallel_loop(0, ECA)
            def _fin(r):
                for j in range(FH // 16):
                    sl = pl.ds(j * 16, 16)
                    slh = pl.ds(FH + j * 16, 16)
                    rcp = 1.0 / (mg[r, sl] + 1e-16)
                    exv0[r, sl] = rcp
                    exv1[r, sl] = rcp * mg[r, slh] + gx0[r, slh]

            pltpu.sync_copy(exv0, rb_hbm.at[pl.ds(rr, ECA)])
            pltpu.sync_copy(exv1, xc1b_hbm.at[pl.ds(rr, ECA)])


def _sc_agg1(x0_pad, esrc, edst, exa, exb):
    f = pl.kernel(
        _agg1_body,
        out_type=(
            jax.ShapeDtypeStruct((NP, FH), _f32),
            jax.ShapeDtypeStruct((NP, FH), _f32),
            jax.ShapeDtypeStruct((NP, FH), _f32),
            jax.ShapeDtypeStruct((NP, FH), _f32),
        ),
        mesh=_mesh(),
        scratch_types=[
            pltpu.VMEM_SHARED((NP, F), _f32),
            pltpu.VMEM((ECA,), _i32),
            pltpu.VMEM((ECA,), _i32),
            pltpu.VMEM((ECA,), _i32),
            pltpu.VMEM((ECA,), _i32),
            pltpu.VMEM((ECA, FH), _f32),
            pltpu.VMEM((ECA, FH), _f32),
            pltpu.VMEM((ECA, F), _f32),
            pltpu.VMEM((ECA, F), _f32),
            pltpu.VMEM((ECA, F), _f32),
        ] + [pltpu.SemaphoreType.DMA] * 8,
    )
    return f(x0_pad, esrc, edst, exa, exb)


def _agg2_body(x0_hbm, xc1_hbm, esrc_hbm, edst_hbm, exa_hbm, exb_hbm,
               ra_hbm, rb_hbm,
               xc2a_hbm, xc2b_hbm,
               acc_sp, si0, si1, di0, di1, exv0, exv1, gx0, gx1, mg, *sems):
    cid = lax.axis_index("c")
    sid = lax.axis_index("s")
    r0 = sid * RT
    si = (si0, si1)
    di = (di0, di1)
    exv = (exv0, exv1)
    gx = (gx0, gx1)

    @plsc.parallel_loop(0, ECA)
    def _z(r):
        for j in range(F // 16):
            mg[r, pl.ds(j * 16, 16)] = jnp.zeros((16,), _f32)

    for b in range(RT // ECA):
        pltpu.sync_copy(mg, acc_sp.at[pl.ds(r0 + b * ECA, ECA)])
    plsc.subcore_barrier()

    def process(cur):
        exc = exv[cur]
        gxc = gx[cur]

        @pl.when(cid == 0)
        def _():
            @plsc.parallel_loop(0, ECA)
            def _mul(r):
                for j in range(FH // 16):
                    sl = pl.ds(j * 16, 16)
                    mg[r, sl] = exc[r, sl] * gxc[r, sl]

        @pl.when(cid == 1)
        def _():
            @plsc.parallel_loop(0, ECA)
            def _mul(r):
                for j in range(FH // 16):
                    sl = pl.ds(j * 16, 16)
                    mg[r, sl] = exc[r, sl] * gxc[r, pl.ds(FH + j * 16, 16)]

        pltpu.sync_copy(mg, acc_sp.at[si[cur]], add=True)

    _edge_stream(cid, sid, esrc_hbm, edst_hbm, exa_hbm, exb_hbm, xc1_hbm,
                 si, di, exv, gx, sems, process)
    plsc.subcore_barrier()

    for b in range(RT // ECA):
        rr = r0 + b * ECA
        pltpu.sync_copy(acc_sp.at[pl.ds(rr, ECA)], mg)
        pltpu.sync_copy(x0_hbm.at[pl.ds(rr, ECA)], gx0)

        @pl.when(cid == 0)
        def _():
            pltpu.sync_copy(ra_hbm.at[pl.ds(rr, ECA)], exv0)

            @plsc.parallel_loop(0, ECA)
            def _fin(r):
                for j in range(FH // 16):
                    sl = pl.ds(j * 16, 16)
                    exv1[r, sl] = exv0[r, sl] * mg[r, sl] + gx0[r, sl]

            pltpu.sync_copy(exv1, xc2a_hbm.at[pl.ds(rr, ECA)])

        @pl.when(cid == 1)
        def _():
            pltpu.sync_copy(rb_hbm.at[pl.ds(rr, ECA)], exv0)

            @plsc.parallel_loop(0, ECA)
            def _fin(r):
                for j in range(FH // 16):
                    sl = pl.ds(j * 16, 16)
                    exv1[r, sl] = (exv0[r, sl] * mg[r, sl]
                                   + gx0[r, pl.ds(FH + j * 16, 16)])

            pltpu.sync_copy(exv1, xc2b_hbm.at[pl.ds(rr, ECA)])


def _sc_agg2(x0_pad, xc1_full, esrc, edst, exa, exb, ra, rb):
    f = pl.kernel(
        _agg2_body,
        out_type=(
            jax.ShapeDtypeStruct((NP, FH), _f32),
            jax.ShapeDtypeStruct((NP, FH), _f32),
        ),
        mesh=_mesh(),
        scratch_types=[
            pltpu.VMEM_SHARED((NP, F), _f32),
            pltpu.VMEM((ECA,), _i32),
            pltpu.VMEM((ECA,), _i32),
            pltpu.VMEM((ECA,), _i32),
            pltpu.VMEM((ECA,), _i32),
            pltpu.VMEM((ECA, FH), _f32),
            pltpu.VMEM((ECA, FH), _f32),
            pltpu.VMEM((ECA, F), _f32),
            pltpu.VMEM((ECA, F), _f32),
            pltpu.VMEM((ECA, F), _f32),
        ] + [pltpu.SemaphoreType.DMA] * 8,
    )
    return f(x0_pad, xc1_full, esrc, edst, exa, exb, ra, rb)


def _mlp_body(xc2a_ref, xc2b_ref, x0_ref, w1_ref, b1_ref, a1_ref, w2_ref,
              b2_ref, a2_ref, w3_ref, b3_ref, out_ref):
    x0 = x0_ref[...]
    w1 = w1_ref[...]
    h = (jnp.dot(xc2a_ref[...], w1[:FH, :], preferred_element_type=_f32)
         + jnp.dot(xc2b_ref[...], w1[FH:F, :], preferred_element_type=_f32)
         + jnp.dot(x0, w1[F:, :], preferred_element_type=_f32)
         + b1_ref[...])
    a1 = a1_ref[...]
    h = jnp.where(h >= 0, h, a1 * h)
    h = jnp.dot(h, w2_ref[...], preferred_element_type=_f32) + b2_ref[...]
    a2 = a2_ref[...]
    h = jnp.where(h >= 0, h, a2 * h)
    lg = jnp.dot(h, w3_ref[...], preferred_element_type=_f32) + b3_ref[...]
    m = jnp.max(lg, axis=1, keepdims=True)
    lse = jnp.log(jnp.sum(jnp.exp(lg - m), axis=1, keepdims=True))
    out_ref[...] = lg - m - lse


def _tc_mlp(xc2a, xc2b, x0, w1, b1, a1, w2, b2, a2, w3, b3):
    return pl.pallas_call(
        _mlp_body,
        out_shape=jax.ShapeDtypeStruct((N, NCLASS), _f32),
    )(xc2a, xc2b, x0, w1, b1, a1, w2, b2, a2, w3, b3)


def kernel(x, x_cov, adj_vals, feature_corr, bn_gamma, bn_beta, p0_W, p0_C,
           p1_W, p1_C, mlp_W1, mlp_b1, mlp_a1, mlp_W2, mlp_b2, mlp_a2,
           mlp_W3, mlp_b3, edge_index):
    edge_index = edge_index.astype(_i32)
    esrc = jnp.pad(edge_index[0], (0, EP - E), constant_values=N)
    edst = jnp.pad(edge_index[1], (0, EP - E), constant_values=N)
    adjp = jnp.pad(adj_vals, (0, EP - E))
    p0_Cp = jnp.pad(p0_C, ((0, CP - C0), (0, 0)))
    x0, s0p, gp, loss = _tc_prep(
        x, x_cov, bn_gamma.reshape(1, F), bn_beta.reshape(1, F), p0_W, p0_Cp,
        feature_corr.reshape(1, F))
    s0p_pad = jnp.pad(s0p, ((0, NP - N), (0, 0)))
    x0_pad = jnp.pad(x0, ((0, NP - N), (0, 0)))
    p = _sc_edges(s0p_pad, esrc, edst)
    exa, exb = _tc_exp(p, gp, adjp[:, None])
    xc1a, xc1b, ra, rb = _sc_agg1(x0_pad, esrc, edst, exa, exb)
    xc1_full = jnp.concatenate([xc1a, xc1b], axis=1)
    xc2a, xc2b = _sc_agg2(x0_pad, xc1_full, esrc, edst, exa, exb, ra, rb)
    out = _tc_mlp(xc2a[:N], xc2b[:N], x0, mlp_W1,
                  mlp_b1.reshape(1, F), mlp_a1.reshape(1, 1), mlp_W2,
                  mlp_b2.reshape(1, F), mlp_a2.reshape(1, 1), mlp_W3,
                  mlp_b3.reshape(1, NCLASS))
    return (out, loss.reshape(()))


# speedup vs baseline: 1.4623x; 1.4623x over previous
import jax
import jax.numpy as jnp
from jax import lax
from jax.experimental import pallas as pl
from jax.experimental.pallas import tpu as pltpu
from jax.experimental.pallas import tpu_sc as plsc

N = 10000
NP = 10240
E = 160000
EP = 163840
F = 128
FH = 64
C0 = 100
CP = 128
NCLASS = 40
NS = 16
NC = 2
ECB = 128
ECA = 64
NCHB = EP // ECB
NCHA = EP // ECA
RT = NP // NS

_f32 = jnp.float32
_i32 = jnp.int32


def _mesh():
    return plsc.VectorSubcoreMesh(core_axis_name="c", subcore_axis_name="s",
                                  num_cores=NC, num_subcores=NS)


def _prep_body(x_ref, xcov_ref, g_ref, b_ref, p0w_ref, p0c_ref, fc_ref,
               x0_ref, s0_ref, gp_ref, loss_ref):
    x = x_ref[...]
    mean = jnp.mean(x, axis=0, keepdims=True)
    var = jnp.mean((x - mean) ** 2, axis=0, keepdims=True)
    x0 = (x - mean) * lax.rsqrt(var + 1e-5) * g_ref[...] + b_ref[...]
    x0_ref[...] = x0

    xcov = xcov_ref[...]
    h0 = jnp.tanh(jnp.dot(xcov, p0w_ref[...], preferred_element_type=_f32))
    logits = lax.dot_general(h0, p0c_ref[...], (((1,), (1,)), ((), ())),
                             preferred_element_type=_f32)
    col = lax.broadcasted_iota(_i32, (N, CP), 1)
    lm = jnp.where(col < C0, logits, -1e30)
    m = jnp.max(lm, axis=1, keepdims=True)
    e = jnp.exp(lm - m)
    s0 = e / jnp.sum(e, axis=1, keepdims=True)
    s0_ref[...] = s0

    ones_n = jnp.ones((N, 1), _f32)
    denom = lax.dot_general(s0, ones_n, (((0,), (0,)), ((), ())),
                            preferred_element_type=_f32)
    sc_t = lax.dot_general(s0, xcov, (((0,), (0,)), ((), ())),
                           preferred_element_type=_f32)
    xc1c = sc_t * (1.0 / (denom + 1e-8))
    xcov2 = jnp.sum(xc1c, axis=0, keepdims=True) / (C0 + 1e-8)
    row = lax.broadcasted_iota(_i32, (CP, F), 0)
    corr1 = jnp.abs(xc1c - xcov2)
    loss_ref[...] = jnp.sum(jnp.where(row < C0, corr1, 0.0),
                            keepdims=True).reshape(1, 1) / (C0 * F)
    z = fc_ref[...] - corr1
    gain = 1.0 + jnp.tanh(0.5 * z)
    g2 = gain * gain
    gp_ref[...] = jnp.where(row < C0, g2, 0.0)


def _tc_prep(x, x_cov, bn_gamma, bn_beta, p0_W, p0_Cp, fc):
    return pl.pallas_call(
        _prep_body,
        out_shape=(
            jax.ShapeDtypeStruct((N, F), _f32),
            jax.ShapeDtypeStruct((N, CP), _f32),
            jax.ShapeDtypeStruct((CP, F), _f32),
            jax.ShapeDtypeStruct((1, 1), _f32),
        ),
    )(x, x_cov, bn_gamma, bn_beta, p0_W, p0_Cp, fc)


def _edges_body(s0_hbm, esrc_hbm, edst_hbm, p_hbm,
                s0_sp, si, di, ga, gb, sem_s, sem_d, sem_1, sem_2):
    cid = lax.axis_index("c")
    sid = lax.axis_index("s")
    wid = cid * NS + sid
    nw = NC * NS
    r0 = sid * RT
    for b in range(RT // ECB):
        rr = r0 + b * ECB
        pltpu.sync_copy(s0_hbm.at[pl.ds(rr, ECB)], ga)
        pltpu.sync_copy(ga, s0_sp.at[pl.ds(rr, ECB)])
    plsc.subcore_barrier()

    def e0(k):
        return pl.multiple_of((wid + nw * k) * ECB, ECB)

    def step(k, carry):
        c1 = pltpu.async_copy(esrc_hbm.at[pl.ds(e0(k), ECB)], si, sem_s)
        c2 = pltpu.async_copy(edst_hbm.at[pl.ds(e0(k), ECB)], di, sem_d)
        c1.wait()
        c2.wait()
        g1 = pltpu.async_copy(s0_sp.at[si], ga, sem_1)
        g2 = pltpu.async_copy(s0_sp.at[di], gb, sem_2)
        g1.wait()
        g2.wait()

        @plsc.parallel_loop(0, ECB)
        def _mul(r):
            for j in range(CP // 16):
                sl = pl.ds(j * 16, 16)
                ga[r, sl] = ga[r, sl] * gb[r, sl]

        pltpu.sync_copy(ga, p_hbm.at[pl.ds(e0(k), ECB)])
        return carry

    lax.fori_loop(0, NCHB // nw, step, 0)


def _sc_edges(s0p, esrc, edst):
    f = pl.kernel(
        _edges_body,
        out_type=jax.ShapeDtypeStruct((EP, CP), _f32),
        mesh=_mesh(),
        scratch_types=[
            pltpu.VMEM_SHARED((NP, CP), _f32),
            pltpu.VMEM((ECB,), _i32),
            pltpu.VMEM((ECB,), _i32),
            pltpu.VMEM((ECB, CP), _f32),
            pltpu.VMEM((ECB, CP), _f32),
        ] + [pltpu.SemaphoreType.DMA] * 4,
    )
    return f(s0p, esrc, edst)


def _exp_body(p_ref, gp_ref, adj_ref, ex_ref):
    v = jnp.dot(p_ref[...], gp_ref[...], preferred_element_type=_f32)
    ex_ref[...] = jnp.exp(v + adj_ref[...])


def _tc_exp(p, gp, adj2d):
    blk = 1280
    return pl.pallas_call(
        _exp_body,
        grid=(EP // blk,),
        in_specs=[
            pl.BlockSpec((blk, CP), lambda i: (i, 0)),
            pl.BlockSpec((CP, F), lambda i: (0, 0)),
            pl.BlockSpec((blk, 1), lambda i: (i, 0)),
        ],
        out_specs=pl.BlockSpec((blk, F), lambda i: (i, 0)),
        out_shape=jax.ShapeDtypeStruct((EP, F), _f32),
    )(p, gp, adj2d)


def _edge_stream(cid, sid, esrc_hbm, edst_hbm, ex_hbm, table_hbm,
                 si, di, exv, gx, sems, process):
    def e0(k):
        return pl.multiple_of((sid + NS * k) * ECA, ECA)

    def step(t, carry):
        k0 = 2 * t
        ld = []
        for u in (0, 1):
            ld.append((
                pltpu.async_copy(
                    esrc_hbm.at[pl.ds(e0(k0 + u), ECA)], si[u], sems[4 * u]),
                pltpu.async_copy(
                    edst_hbm.at[pl.ds(e0(k0 + u), ECA)], di[u],
                    sems[4 * u + 1]),
                pltpu.async_copy(
                    ex_hbm.at[pl.ds(e0(k0 + u), ECA)], exv[u],
                    sems[4 * u + 2]),
            ))

        gs = []
        for u in (0, 1):
            ld[u][0].wait()
            ld[u][1].wait()
            gs.append(pltpu.async_copy(table_hbm.at[di[u]], gx[u],
                                       sems[4 * u + 3]))

        for u in (0, 1):
            ld[u][2].wait()
            gs[u].wait()
            process(u)

        return carry

    lax.fori_loop(0, NCHA // NS // 2, step, 0)


def _agg1_body(x0_hbm, esrc_hbm, edst_hbm, ex_hbm,
               ya_hbm, yb_hbm,
               acc_sp, si0, si1, di0, di1, exv0, exv1, gx0, gx1, mg,
               *sems):
    cid = lax.axis_index("c")
    sid = lax.axis_index("s")
    r0 = sid * RT
    si = (si0, si1)
    di = (di0, di1)
    exv = (exv0, exv1)
    gx = (gx0, gx1)

    @plsc.parallel_loop(0, ECA)
    def _z(r):
        for j in range(F // 16):
            mg[r, pl.ds(j * 16, 16)] = jnp.zeros((16,), _f32)

    for b in range(RT // ECA):
        pltpu.sync_copy(mg, acc_sp.at[pl.ds(r0 + b * ECA, ECA)])
    plsc.subcore_barrier()

    def process(cur):
        exc = exv[cur]
        gxc = gx[cur]

        @pl.when(cid == 0)
        def _():
            @plsc.parallel_loop(0, ECA)
            def _mul(r):
                for j in range(FH // 16):
                    sl = pl.ds(j * 16, 16)
                    v = exc[r, sl]
                    mg[r, sl] = v
                    mg[r, pl.ds(FH + j * 16, 16)] = v * gxc[r, sl]

        @pl.when(cid == 1)
        def _():
            @plsc.parallel_loop(0, ECA)
            def _mul(r):
                for j in range(FH // 16):
                    sl = pl.ds(j * 16, 16)
                    slh = pl.ds(FH + j * 16, 16)
                    v = exc[r, slh]
                    mg[r, sl] = v
                    mg[r, slh] = v * gxc[r, slh]

        pltpu.sync_copy(mg, acc_sp.at[si[cur]], add=True)

    _edge_stream(cid, sid, esrc_hbm, edst_hbm, ex_hbm, x0_hbm,
                 si, di, exv, gx, sems, process)
    plsc.subcore_barrier()

    for b in range(RT // ECA):
        rr = r0 + b * ECA
        pltpu.sync_copy(acc_sp.at[pl.ds(rr, ECA)], mg)
        pltpu.sync_copy(x0_hbm.at[pl.ds(rr, ECA)], gx0)

        @pl.when(cid == 0)
        def _():
            @plsc.parallel_loop(0, ECA)
            def _fin(r):
                for j in range(FH // 16):
                    sl = pl.ds(j * 16, 16)
                    slh = pl.ds(FH + j * 16, 16)
                    rcp = 1.0 / (mg[r, sl] + 1e-16)
                    mg[r, sl] = rcp
                    mg[r, slh] = rcp * mg[r, slh] + gx0[r, sl]

            pltpu.sync_copy(mg, ya_hbm.at[pl.ds(rr, ECA)])

        @pl.when(cid == 1)
        def _():
            @plsc.parallel_loop(0, ECA)
            def _fin(r):
                for j in range(FH // 16):
                    sl = pl.ds(j * 16, 16)
                    slh = pl.ds(FH + j * 16, 16)
                    rcp = 1.0 / (mg[r, sl] + 1e-16)
                    mg[r, sl] = rcp
                    mg[r, slh] = rcp * mg[r, slh] + gx0[r, slh]

            pltpu.sync_copy(mg, yb_hbm.at[pl.ds(rr, ECA)])


def _sc_agg1(x0_pad, esrc, edst, ex):
    f = pl.kernel(
        _agg1_body,
        out_type=(
            jax.ShapeDtypeStruct((NP, F), _f32),
            jax.ShapeDtypeStruct((NP, F), _f32),
        ),
        mesh=_mesh(),
        scratch_types=[
            pltpu.VMEM_SHARED((NP, F), _f32),
            pltpu.VMEM((ECA,), _i32),
            pltpu.VMEM((ECA,), _i32),
            pltpu.VMEM((ECA,), _i32),
            pltpu.VMEM((ECA,), _i32),
            pltpu.VMEM((ECA, F), _f32),
            pltpu.VMEM((ECA, F), _f32),
            pltpu.VMEM((ECA, F), _f32),
            pltpu.VMEM((ECA, F), _f32),
            pltpu.VMEM((ECA, F), _f32),
        ] + [pltpu.SemaphoreType.DMA] * 8,
    )
    return f(x0_pad, esrc, edst, ex)


def _agg2_body(xc1_hbm, esrc_hbm, edst_hbm, ex_hbm, t2p_hbm,
               acc_sp, si0, si1, di0, di1, exv0, exv1, gx0, gx1, mg,
               *sems):
    cid = lax.axis_index("c")
    sid = lax.axis_index("s")
    r0 = sid * RT
    si = (si0, si1)
    di = (di0, di1)
    exv = (exv0, exv1)
    gx = (gx0, gx1)

    @plsc.parallel_loop(0, ECA)
    def _z(r):
        for j in range(F // 16):
            mg[r, pl.ds(j * 16, 16)] = jnp.zeros((16,), _f32)

    for b in range(RT // ECA):
        pltpu.sync_copy(mg, acc_sp.at[pl.ds(r0 + b * ECA, ECA)])
    plsc.subcore_barrier()

    base = cid * (NCHA // NC)

    def e0(k):
        return pl.multiple_of((base + sid + NS * k) * ECA, ECA)

    def step(t, carry):
        k0 = 2 * t
        ld = []
        for u in (0, 1):
            ld.append((
                pltpu.async_copy(
                    esrc_hbm.at[pl.ds(e0(k0 + u), ECA)], si[u], sems[4 * u]),
                pltpu.async_copy(
                    edst_hbm.at[pl.ds(e0(k0 + u), ECA)], di[u],
                    sems[4 * u + 1]),
                pltpu.async_copy(
                    ex_hbm.at[pl.ds(e0(k0 + u), ECA)], exv[u],
                    sems[4 * u + 2]),
            ))

        gs = []
        for u in (0, 1):
            ld[u][0].wait()
            ld[u][1].wait()
            gs.append(pltpu.async_copy(xc1_hbm.at[di[u]], gx[u],
                                       sems[4 * u + 3]))

        for u in (0, 1):
            ld[u][2].wait()
            gs[u].wait()
            exc = exv[u]
            gxc = gx[u]

            @plsc.parallel_loop(0, ECA)
            def _mul(r):
                for j in range(F // 16):
                    sl = pl.ds(j * 16, 16)
                    mg[r, sl] = exc[r, sl] * gxc[r, sl]

            pltpu.sync_copy(mg, acc_sp.at[si[u]], add=True)

        return carry

    lax.fori_loop(0, NCHA // NC // NS // 2, step, 0)
    plsc.subcore_barrier()

    for b in range(RT // ECA):
        rr = r0 + b * ECA
        pltpu.sync_copy(acc_sp.at[pl.ds(rr, ECA)], mg)

        @pl.when(cid == 0)
        def _():
            pltpu.sync_copy(mg, t2p_hbm.at[0, pl.ds(rr, ECA)])

        @pl.when(cid == 1)
        def _():
            pltpu.sync_copy(mg, t2p_hbm.at[1, pl.ds(rr, ECA)])


def _sc_agg2(xc1_full, esrc, edst, ex):
    f = pl.kernel(
        _agg2_body,
        out_type=jax.ShapeDtypeStruct((NC, NP, F), _f32),
        mesh=_mesh(),
        scratch_types=[
            pltpu.VMEM_SHARED((NP, F), _f32),
            pltpu.VMEM((ECA,), _i32),
            pltpu.VMEM((ECA,), _i32),
            pltpu.VMEM((ECA,), _i32),
            pltpu.VMEM((ECA,), _i32),
            pltpu.VMEM((ECA, F), _f32),
            pltpu.VMEM((ECA, F), _f32),
            pltpu.VMEM((ECA, F), _f32),
            pltpu.VMEM((ECA, F), _f32),
            pltpu.VMEM((ECA, F), _f32),
        ] + [pltpu.SemaphoreType.DMA] * 8,
    )
    return f(xc1_full, esrc, edst, ex)


def _mlp_body(t20_ref, t21_ref, ra_ref, rb_ref, x0_ref, w1_ref, b1_ref,
              a1_ref, w2_ref, b2_ref, a2_ref, w3_ref, b3_ref, out_ref):
    x0 = x0_ref[...]
    w1 = w1_ref[...]
    t2 = t20_ref[...] + t21_ref[...]
    xc2a = ra_ref[...] * t2[:, :FH] + x0[:, :FH]
    xc2b = rb_ref[...] * t2[:, FH:] + x0[:, FH:]
    h = (jnp.dot(xc2a, w1[:FH, :], preferred_element_type=_f32)
         + jnp.dot(xc2b, w1[FH:F, :], preferred_element_type=_f32)
         + jnp.dot(x0, w1[F:, :], preferred_element_type=_f32)
         + b1_ref[...])
    a1 = a1_ref[...]
    h = jnp.where(h >= 0, h, a1 * h)
    h = jnp.dot(h, w2_ref[...], preferred_element_type=_f32) + b2_ref[...]
    a2 = a2_ref[...]
    h = jnp.where(h >= 0, h, a2 * h)
    lg = jnp.dot(h, w3_ref[...], preferred_element_type=_f32) + b3_ref[...]
    m = jnp.max(lg, axis=1, keepdims=True)
    lse = jnp.log(jnp.sum(jnp.exp(lg - m), axis=1, keepdims=True))
    out_ref[...] = lg - m - lse


def _tc_mlp(t20, t21, ra, rb, x0, w1, b1, a1, w2, b2, a2, w3, b3):
    return pl.pallas_call(
        _mlp_body,
        out_shape=jax.ShapeDtypeStruct((N, NCLASS), _f32),
    )(t20, t21, ra, rb, x0, w1, b1, a1, w2, b2, a2, w3, b3)


def kernel(x, x_cov, adj_vals, feature_corr, bn_gamma, bn_beta, p0_W, p0_C,
           p1_W, p1_C, mlp_W1, mlp_b1, mlp_a1, mlp_W2, mlp_b2, mlp_a2,
           mlp_W3, mlp_b3, edge_index):
    edge_index = edge_index.astype(_i32)
    esrc = jnp.pad(edge_index[0], (0, EP - E), constant_values=N)
    edst = jnp.pad(edge_index[1], (0, EP - E), constant_values=N)
    adjp = jnp.pad(adj_vals, (0, EP - E))
    p0_Cp = jnp.pad(p0_C, ((0, CP - C0), (0, 0)))
    x0, s0p, gp, loss = _tc_prep(
        x, x_cov, bn_gamma.reshape(1, F), bn_beta.reshape(1, F), p0_W, p0_Cp,
        feature_corr.reshape(1, F))
    s0p_pad = jnp.pad(s0p, ((0, NP - N), (0, 0)))
    x0_pad = jnp.pad(x0, ((0, NP - N), (0, 0)))
    p = _sc_edges(s0p_pad, esrc, edst)
    ex = _tc_exp(p, gp, adjp[:, None])
    ya, yb = _sc_agg1(x0_pad, esrc, edst, ex)
    xc1_full = jnp.concatenate([ya[:, FH:], yb[:, FH:]], axis=1)
    t2p = _sc_agg2(xc1_full, esrc, edst, ex)
    out = _tc_mlp(t2p[0, :N], t2p[1, :N], ya[:N, :FH], yb[:N, :FH], x0,
                  mlp_W1, mlp_b1.reshape(1, F), mlp_a1.reshape(1, 1),
                  mlp_W2, mlp_b2.reshape(1, F), mlp_a2.reshape(1, 1),
                  mlp_W3, mlp_b3.reshape(1, NCLASS))
    return (out, loss.reshape(()))
